# baseline (speedup 1.0000x reference)
import jax
import jax.numpy as jnp
from jax.experimental import pallas as pl
from jax.experimental.pallas import tpu as pltpu

TILE = 1024
NSPLIT = 4


def _router_kernel(*refs):
    x_refs = refs[:NSPLIT]
    w_hbm, bias_ref, out_ref, wbuf, wsem = refs[NSPLIT:]
    i = pl.program_id(0)
    q = x_refs[0].shape[1]

    @pl.when(i == 0)
    def _():
        pltpu.make_async_copy(w_hbm, wbuf, wsem).start()
        pltpu.make_async_copy(w_hbm, wbuf, wsem).wait()

    logits = bias_ref[...]
    for k in range(NSPLIT):
        logits = logits + jax.lax.dot_general(
            x_refs[k][...], wbuf[:, k * q:(k + 1) * q],
            dimension_numbers=(((1,), (1,)), ((), ())),
            preferred_element_type=jnp.float32,
        )
    m = jnp.max(logits, axis=-1, keepdims=True)
    e = jnp.exp(logits - m)
    out_ref[...] = e / jnp.sum(e, axis=-1, keepdims=True)


@jax.jit
def kernel(x, W, bias):
    n_tokens, dim = x.shape
    n_experts = W.shape[0]
    q = dim // NSPLIT
    grid = (n_tokens // TILE,)

    def mk(k):
        return pl.BlockSpec((TILE, q), lambda i, k=k: (i, k))

    return pl.pallas_call(
        _router_kernel,
        grid=grid,
        in_specs=[mk(k) for k in range(NSPLIT)]
        + [
            pl.BlockSpec(memory_space=pltpu.MemorySpace.HBM),
            pl.BlockSpec((1, n_experts), lambda i: (0, 0)),
        ],
        out_specs=pl.BlockSpec((TILE, n_experts), lambda i: (i, 0)),
        out_shape=jax.ShapeDtypeStruct((n_tokens, n_experts), jnp.float32),
        scratch_shapes=[
            pltpu.VMEM((n_experts, dim), jnp.float32),
            pltpu.SemaphoreType.DMA,
        ],
        compiler_params=pltpu.CompilerParams(
            dimension_semantics=("arbitrary",),
        ),
    )(*([x] * NSPLIT), W, bias.reshape(1, n_experts))


# speedup vs baseline: 1.0052x; 1.0052x over previous
import jax
import jax.numpy as jnp
from jax.experimental import pallas as pl
from jax.experimental.pallas import tpu as pltpu

TILE = 1024
NBUF = 3
NSPLIT = 4


def _router_kernel(x_hbm, w_ref, bias_ref, out_ref, xbuf, sems):
    n_tiles = pl.num_programs(0)
    i = pl.program_id(0)
    dim = x_hbm.shape[1]
    q = dim // NSPLIT

    def copy(t, slot, k):
        return pltpu.make_async_copy(
            x_hbm.at[pl.ds(t * TILE, TILE), pl.ds(k * q, q)],
            xbuf.at[slot, :, pl.ds(k * q, q)],
            sems.at[slot, k],
        )

    def start(t, slot):
        for k in range(NSPLIT):
            copy(t, slot, k).start()

    @pl.when(i == 0)
    def _():
        for t in range(NBUF - 1):
            start(t, t)

    nxt = i + NBUF - 1
    @pl.when(nxt < n_tiles)
    def _():
        start(nxt, jax.lax.rem(nxt, NBUF))

    slot = jax.lax.rem(i, NBUF)
    logits = bias_ref[...]
    for k in range(NSPLIT):
        copy(i, slot, k).wait()
        logits = logits + jax.lax.dot_general(
            xbuf[slot, :, k * q:(k + 1) * q], w_ref[:, k * q:(k + 1) * q],
            dimension_numbers=(((1,), (1,)), ((), ())),
            preferred_element_type=jnp.float32,
        )
    m = jnp.max(logits, axis=-1, keepdims=True)
    e = jnp.exp(logits - m)
    out_ref[...] = e / jnp.sum(e, axis=-1, keepdims=True)


@jax.jit
def kernel(x, W, bias):
    n_tokens, dim = x.shape
    n_experts = W.shape[0]
    grid = (n_tokens // TILE,)
    return pl.pallas_call(
        _router_kernel,
        grid=grid,
        in_specs=[
            pl.BlockSpec(memory_space=pltpu.MemorySpace.HBM),
            pl.BlockSpec((n_experts, dim), lambda i: (0, 0)),
            pl.BlockSpec((1, n_experts), lambda i: (0, 0)),
        ],
        out_specs=pl.BlockSpec((TILE, n_experts), lambda i: (i, 0)),
        out_shape=jax.ShapeDtypeStruct((n_tokens, n_experts), jnp.float32),
        scratch_shapes=[
            pltpu.VMEM((NBUF, TILE, dim), jnp.float32),
            pltpu.SemaphoreType.DMA((NBUF, NSPLIT)),
        ],
        compiler_params=pltpu.CompilerParams(
            dimension_semantics=("arbitrary",),
        ),
    )(x, W, bias.reshape(1, n_experts))
